# Initial kernel scaffold; baseline (speedup 1.0000x reference)
import jax
import jax.numpy as jnp
from jax.experimental import pallas as pl


def kernel(x, edge_index, batch, W1, b1, ln1_g, ln1_b, W2, b2, Wa, ba, ln2_g, ln2_b, Wb, bb):
    raise NotImplementedError("write your pallas kernel here")


# speedup vs baseline: 18.1387x; 18.1387x over previous
import functools

import jax
import jax.numpy as jnp
from jax import lax
from jax.experimental import pallas as pl
from jax.experimental.pallas import tpu as pltpu
from jax.experimental.pallas import tpu_sc as plsc

N = 10000
E = 320000
D = 128
H = 128
C = 10
G = 64
EPS = 1e-5

NC = 2
NS = 16
NW = NC * NS
EPW = E // NW
CHUNK = 80
NSTEP = EPW // CHUNK
RPS = N // NS

def _mesh():
    return plsc.VectorSubcoreMesh(core_axis_name="c", subcore_axis_name="s")


def _zero_vmem2d(ref, nrows):
    def body(i, _):
        for j in range(8):
            ref[i, pl.ds(j * 16, 16)] = jnp.zeros((16,), jnp.float32)
        return 0
    lax.fori_loop(0, nrows, body, 0)


def _hist_body(col_hbm, out_hbm, idx_v, ones_v, zbuf_v, acc_sh, sem):
    c = lax.axis_index("c")
    s = lax.axis_index("s")
    wid = s * NC + c

    for j in range(CHUNK // 16):
        ones_v[pl.ds(j * 16, 16)] = jnp.ones((16,), jnp.float32)
    def zb(i, _):
        zbuf_v[pl.ds(i * 16, 16)] = jnp.zeros((16,), jnp.float32)
        return 0
    lax.fori_loop(0, 62, zb, 0)
    zbuf_v[pl.ds(984, 16)] = jnp.zeros((16,), jnp.float32)

    @pl.when(s < 10)
    def _():
        pltpu.sync_copy(zbuf_v, acc_sh.at[pl.ds(s * 1000, 1000)])
    plsc.subcore_barrier()

    def step(t, _):
        base = wid * EPW + t * CHUNK
        pltpu.sync_copy(col_hbm.at[pl.ds(base, CHUNK)], idx_v)
        pltpu.sync_copy(ones_v, acc_sh.at[idx_v], add=True)
        return 0
    lax.fori_loop(0, NSTEP, step, 0)
    plsc.subcore_barrier()

    @pl.when(s < 10)
    def _():
        pltpu.sync_copy(acc_sh.at[pl.ds(s * 1000, 1000)], zbuf_v)
        pltpu.sync_copy(zbuf_v, out_hbm.at[pl.ds(c * N + s * 1000, 1000)])


def _hist_sc(col):
    k = pl.kernel(
        _hist_body,
        mesh=_mesh(),
        out_type=jax.ShapeDtypeStruct((2 * N,), jnp.float32),
        scratch_types=[
            pltpu.VMEM((CHUNK,), jnp.int32),
            pltpu.VMEM((CHUNK,), jnp.float32),
            pltpu.VMEM((1000,), jnp.float32),
            pltpu.VMEM_SHARED((N,), jnp.float32),
            pltpu.SemaphoreType.DMA,
        ],
    )
    return k(col)


def _spmm_body(row_hbm, col_hbm, y_hbm, out_hbm,
               idxr_v, idxc_v, rows_v, zbuf_v, acc_sh, sem):
    c = lax.axis_index("c")
    s = lax.axis_index("s")
    wid = s * NC + c

    _zero_vmem2d(zbuf_v, 200)
    @pl.when(s < 10)
    def _():
        for k in range(5):
            pltpu.sync_copy(zbuf_v, acc_sh.at[pl.ds(s * 1000 + k * 200, 200)])
    plsc.subcore_barrier()

    def step(t, _):
        base = wid * EPW + t * CHUNK
        pltpu.sync_copy(row_hbm.at[pl.ds(base, CHUNK)], idxr_v)
        pltpu.sync_copy(col_hbm.at[pl.ds(base, CHUNK)], idxc_v)
        pltpu.async_copy(y_hbm.at[idxr_v], rows_v, sem).wait()
        pltpu.sync_copy(rows_v, acc_sh.at[idxc_v], add=True)
        return 0
    lax.fori_loop(0, NSTEP, step, 0)
    plsc.subcore_barrier()

    @pl.when(s < 10)
    def _():
        for k in range(5):
            r0 = s * 1000 + k * 200
            pltpu.sync_copy(acc_sh.at[pl.ds(r0, 200)], zbuf_v)
            pltpu.sync_copy(zbuf_v, out_hbm.at[pl.ds(c * N + r0, 200)])


def _spmm_sc(row, col, y):
    k = pl.kernel(
        _spmm_body,
        mesh=_mesh(),
        out_type=jax.ShapeDtypeStruct((2 * N, H), jnp.float32),
        scratch_types=[
            pltpu.VMEM((CHUNK,), jnp.int32),
            pltpu.VMEM((CHUNK,), jnp.int32),
            pltpu.VMEM((CHUNK, H), jnp.float32),
            pltpu.VMEM((200, H), jnp.float32),
            pltpu.VMEM_SHARED((N, H), jnp.float32),
            pltpu.SemaphoreType.DMA,
        ],
    )
    return k(row, col, y)


_BR = 1000
_NB = N // _BR


def _dot(a, b):
    return jnp.dot(a, b, preferred_element_type=jnp.float32,
                   precision=lax.Precision.HIGHEST)


def _tc1_body(x_ref, w1_ref, p0_ref, p1_ref, y1_ref, xw1_ref):
    xw = _dot(x_ref[...], w1_ref[...])
    hist = p0_ref[0, 0, :] + p1_ref[0, 0, :]
    dis = lax.rsqrt(hist + 2.0)
    xw1_ref[...] = xw
    y1_ref[...] = xw * dis[:, None]


def _tc2_body(s0_ref, s1_ref, xw1_ref, p0_ref, p1_ref, b1_ref, g1_ref,
              bb1_ref, w2_ref, y2_ref, xw2_ref):
    hist = p0_ref[0, 0, :] + p1_ref[0, 0, :]
    dis1 = lax.rsqrt(hist + 2.0)
    dis2 = lax.rsqrt(hist + 1.0)
    srow = s0_ref[...] + s1_ref[...]
    pre = (srow * dis1[:, None]
           + (2.0 * dis1 * dis1)[:, None] * xw1_ref[...]
           + b1_ref[...])
    mu = jnp.mean(pre, axis=-1, keepdims=True)
    dvar = pre - mu
    var = jnp.mean(dvar * dvar, axis=-1, keepdims=True)
    t = dvar * lax.rsqrt(var + EPS) * g1_ref[...] + bb1_ref[...]
    h1 = jnp.where(t >= 0, t, 0.2 * t)
    xw2 = _dot(h1, w2_ref[...])
    xw2_ref[...] = xw2
    y2_ref[...] = xw2 * dis2[:, None]


def _tc3_body(s0_ref, s1_ref, xw2_ref, p0_ref, p1_ref, b2_ref, batch_ref,
              wa_ref, ba_ref, g2_ref, bb2_ref, wb_ref, bbb_ref,
              out_ref, pooled_ref):
    i = pl.program_id(0)

    @pl.when(i == 0)
    def _():
        pooled_ref[...] = jnp.full((G, H), -1e30, jnp.float32)

    hist = p0_ref[0, 0, :] + p1_ref[0, 0, :]
    dis2 = lax.rsqrt(hist + 1.0)
    srow = s0_ref[...] + s1_ref[...]
    t = (srow * dis2[:, None]
         + (dis2 * dis2)[:, None] * xw2_ref[...]
         + b2_ref[...])
    h2 = jnp.where(t >= 0, t, 0.2 * t)

    b2d = batch_ref[...]
    g_lo = b2d[0, 0]
    g_hi = b2d[_BR - 1, 0]

    def seg(g, _):
        vals = jnp.where(b2d == g, h2, -1e30)
        m = jnp.max(vals, axis=0)
        cur = pooled_ref[pl.ds(g, 1), :]
        pooled_ref[pl.ds(g, 1), :] = jnp.maximum(cur, m[None, :])
        return 0
    lax.fori_loop(g_lo, g_hi + 1, seg, 0)

    @pl.when(i == _NB - 1)
    def _():
        pooled = pooled_ref[...]
        z = _dot(pooled, wa_ref[...]) + ba_ref[...]
        mu = jnp.mean(z, axis=-1, keepdims=True)
        dv = z - mu
        var = jnp.mean(dv * dv, axis=-1, keepdims=True)
        z = dv * lax.rsqrt(var + EPS) * g2_ref[...] + bb2_ref[...]
        z = jnp.where(z >= 0, z, 0.2 * z)
        z2 = _dot(z, wb_ref[...]) + bbb_ref[...]
        colid = lax.broadcasted_iota(jnp.int32, (G, 128), 1)
        valid = colid < C
        zm = jnp.where(valid, z2, -1e30)
        m = jnp.max(zm, axis=-1, keepdims=True)
        e = jnp.where(valid, jnp.exp(zm - m), 0.0)
        out_ref[...] = e / jnp.sum(e, axis=-1, keepdims=True)


def _row_spec():
    return pl.BlockSpec((_BR, H), lambda i: (i, 0))


def _p_spec():
    return pl.BlockSpec((1, 1, _BR), lambda i: (i, 0, 0))


def _full2(shape):
    return pl.BlockSpec(shape, lambda i: (0,) * len(shape))


def kernel(x, edge_index, batch, W1, b1, ln1_g, ln1_b, W2, b2, Wa, ba,
           ln2_g, ln2_b, Wb, bb):
    row = edge_index[0]
    col = edge_index[1]

    hist2 = _hist_sc(col)
    p3 = hist2.reshape(2, _NB, _BR).transpose(1, 0, 2)
    p0 = p3[:, 0:1, :]
    p1 = p3[:, 1:2, :]

    y1, xw1 = pl.pallas_call(
        _tc1_body,
        grid=(_NB,),
        in_specs=[_row_spec(), _full2((D, H)), _p_spec(), _p_spec()],
        out_specs=[_row_spec(), _row_spec()],
        out_shape=[jax.ShapeDtypeStruct((N, H), jnp.float32)] * 2,
    )(x, W1, p0, p1)

    s1p = _spmm_sc(row, col, y1)

    b1r = b1.reshape(1, H)
    g1r = ln1_g.reshape(1, H)
    bb1r = ln1_b.reshape(1, H)
    y2, xw2 = pl.pallas_call(
        _tc2_body,
        grid=(_NB,),
        in_specs=[_row_spec(), _row_spec(), _row_spec(), _p_spec(), _p_spec(),
                  _full2((1, H)), _full2((1, H)), _full2((1, H)),
                  _full2((H, H))],
        out_specs=[_row_spec(), _row_spec()],
        out_shape=[jax.ShapeDtypeStruct((N, H), jnp.float32)] * 2,
    )(s1p[:N], s1p[N:], xw1, p0, p1, b1r, g1r, bb1r, W2)

    s2p = _spmm_sc(row, col, y2)

    batch3 = jnp.broadcast_to(batch[:, None], (N, 128))
    Wbp = jnp.pad(Wb, ((0, 0), (0, 128 - C)))
    bbp = jnp.pad(bb, (0, 128 - C)).reshape(1, 128)
    out = pl.pallas_call(
        _tc3_body,
        grid=(_NB,),
        in_specs=[_row_spec(), _row_spec(), _row_spec(), _p_spec(), _p_spec(),
                  _full2((1, H)),
                  pl.BlockSpec((_BR, 128), lambda i: (i, 0)),
                  _full2((H, 768)), _full2((1, 768)), _full2((1, 768)),
                  _full2((1, 768)), _full2((768, 128)), _full2((1, 128))],
        out_specs=pl.BlockSpec((G, 128), lambda i: (0, 0)),
        out_shape=jax.ShapeDtypeStruct((G, 128), jnp.float32),
        scratch_shapes=[pltpu.VMEM((G, H), jnp.float32)],
    )(s2p[:N], s2p[N:], xw2, p0, p1, b2.reshape(1, H), batch3,
      Wa, ba.reshape(1, 768), ln2_g.reshape(1, 768), ln2_b.reshape(1, 768),
      Wbp, bbp)
    return out[:, :C]


# speedup vs baseline: 33.2894x; 1.8353x over previous
import functools

import jax
import jax.numpy as jnp
from jax import lax
from jax.experimental import pallas as pl
from jax.experimental.pallas import tpu as pltpu
from jax.experimental.pallas import tpu_sc as plsc

N = 10000
E = 320000
D = 128
H = 128
C = 10
G = 64
EPS = 1e-5

NC = 2
NS = 16
NW = NC * NS
EPW = E // NW
CHUNK = 40
NSTEP = EPW // CHUNK
NBK = 10
NOUT = NSTEP // NBK
NB = 5
KD = 3

def _mesh():
    return plsc.VectorSubcoreMesh(core_axis_name="c", subcore_axis_name="s")


def _zero_vmem2d(ref, nrows):
    def body(i, _):
        for j in range(8):
            ref[i, pl.ds(j * 16, 16)] = jnp.zeros((16,), jnp.float32)
        return 0
    lax.fori_loop(0, nrows, body, 0)


def _hist_body(col_hbm, out_hbm, idx_v, ones_v, zbuf_v, acc_sh, sem):
    c = lax.axis_index("c")
    s = lax.axis_index("s")
    wid = s * NC + c

    for j in range(CHUNK // 16):
        ones_v[pl.ds(j * 16, 16)] = jnp.ones((16,), jnp.float32)
    if CHUNK % 16:
        ones_v[pl.ds(CHUNK - 16, 16)] = jnp.ones((16,), jnp.float32)
    def zb(i, _):
        zbuf_v[pl.ds(i * 16, 16)] = jnp.zeros((16,), jnp.float32)
        return 0
    lax.fori_loop(0, 62, zb, 0)
    zbuf_v[pl.ds(984, 16)] = jnp.zeros((16,), jnp.float32)

    @pl.when(s < 10)
    def _():
        pltpu.sync_copy(zbuf_v, acc_sh.at[pl.ds(s * 1000, 1000)])
    plsc.subcore_barrier()

    def step(t, _):
        base = wid * EPW + t * CHUNK
        pltpu.sync_copy(col_hbm.at[pl.ds(base, CHUNK)], idx_v)
        pltpu.sync_copy(ones_v, acc_sh.at[idx_v], add=True)
        return 0
    lax.fori_loop(0, NSTEP, step, 0)
    plsc.subcore_barrier()

    @pl.when(s < 10)
    def _():
        pltpu.sync_copy(acc_sh.at[pl.ds(s * 1000, 1000)], zbuf_v)
        pltpu.sync_copy(zbuf_v, out_hbm.at[pl.ds(c * N + s * 1000, 1000)])


def _hist_sc(col):
    k = pl.kernel(
        _hist_body,
        mesh=_mesh(),
        out_type=jax.ShapeDtypeStruct((2 * N,), jnp.float32),
        scratch_types=[
            pltpu.VMEM((CHUNK,), jnp.int32),
            pltpu.VMEM((CHUNK,), jnp.float32),
            pltpu.VMEM((1000,), jnp.float32),
            pltpu.VMEM_SHARED((N,), jnp.float32),
            pltpu.SemaphoreType.DMA,
        ],
    )
    return k(col)


NI = 10


def _spmm_body(row_hbm, col_hbm, y_hbm, out_hbm, bufs, acc_sh, *refs):
    c = lax.axis_index("c")
    s = lax.axis_index("s")
    wid = s * NC + c
    rv = refs[:NI]
    cv = refs[NI:2 * NI]
    isem = refs[2 * NI:3 * NI]
    gsem = refs[3 * NI:3 * NI + NB]
    ssem = refs[3 * NI + NB:3 * NI + 2 * NB]
    ebase = wid * EPW

    def bufref(b):
        return bufs.at[pl.ds(b * CHUNK, CHUNK)]

    def idx_load(t, j):
        pltpu.async_copy(row_hbm.at[pl.ds(ebase + t * CHUNK, CHUNK)],
                         rv[j], isem[j])
        pltpu.async_copy(col_hbm.at[pl.ds(ebase + t * CHUNK, CHUNK)],
                         cv[j], isem[j])

    def idx_wait(j):
        pltpu.make_async_copy(row_hbm.at[pl.ds(0, CHUNK)], rv[j],
                              isem[j]).wait()
        pltpu.make_async_copy(col_hbm.at[pl.ds(0, CHUNK)], cv[j],
                              isem[j]).wait()

    def g_start(j, b):
        pltpu.async_copy(y_hbm.at[rv[j]], bufref(b), gsem[b])

    def g_wait(b):
        pltpu.make_async_copy(y_hbm.at[rv[0]], bufref(b), gsem[b]).wait()

    def s_start(j, b):
        pltpu.async_copy(bufref(b), acc_sh.at[cv[j]], ssem[b], add=True)

    def s_wait(b):
        pltpu.make_async_copy(bufref(b), acc_sh.at[cv[0]], ssem[b]).wait()

    for t in range(NI - 3):
        idx_load(t, t)
    _zero_vmem2d(bufs, NB * CHUNK)
    @pl.when(s < 10)
    def _():
        for k in range(5):
            pltpu.sync_copy(bufs, acc_sh.at[pl.ds(s * 1000 + k * 200, 200)])
    plsc.subcore_barrier()

    for t in range(KD):
        idx_wait(t)
        g_start(t, t)

    def outer(i, _):
        for b in range(NBK):
            t = i * NBK + b
            bb = b % NB
            bg = (b + KD) % NB

            @pl.when(t + NI - 3 < NSTEP)
            def _():
                idx_load(t + NI - 3, (b + NI - 3) % NI)

            @pl.when(t >= NB - KD)
            def _():
                s_wait(bg)

            @pl.when(t + KD < NSTEP)
            def _():
                idx_wait((b + KD) % NI)
                g_start((b + KD) % NI, bg)

            g_wait(bb)
            s_start(b, bb)
        return 0
    lax.fori_loop(0, NOUT, outer, 0)

    for t in range(NSTEP - (NB - KD), NSTEP):
        s_wait(t % NB)
    plsc.subcore_barrier()

    @pl.when(s < 10)
    def _():
        for k in range(5):
            r0 = s * 1000 + k * 200
            pltpu.sync_copy(acc_sh.at[pl.ds(r0, 200)], bufs)
            pltpu.sync_copy(bufs, out_hbm.at[pl.ds(c * N + r0, 200)])


def _spmm_sc(row, col, y):
    k = pl.kernel(
        _spmm_body,
        mesh=_mesh(),
        out_type=jax.ShapeDtypeStruct((2 * N, H), jnp.float32),
        scratch_types=[
            pltpu.VMEM((NB * CHUNK, H), jnp.float32),
            pltpu.VMEM_SHARED((N, H), jnp.float32),
        ] + [pltpu.VMEM((CHUNK,), jnp.int32)] * (2 * NI)
          + [pltpu.SemaphoreType.DMA] * (NI + 2 * NB),
    )
    return k(row, col, y)


_BR = 1000
_NB = N // _BR


def _dot(a, b):
    return jnp.dot(a, b, preferred_element_type=jnp.float32,
                   precision=lax.Precision.HIGHEST)


def _tc1_body(x_ref, w1_ref, p0_ref, p1_ref, y1_ref, xw1_ref):
    xw = _dot(x_ref[...], w1_ref[...])
    hist = p0_ref[0, 0, :] + p1_ref[0, 0, :]
    dis = lax.rsqrt(hist + 2.0)
    xw1_ref[...] = xw
    y1_ref[...] = xw * dis[:, None]


def _tc2_body(s0_ref, s1_ref, xw1_ref, p0_ref, p1_ref, b1_ref, g1_ref,
              bb1_ref, w2_ref, y2_ref, xw2_ref):
    hist = p0_ref[0, 0, :] + p1_ref[0, 0, :]
    dis1 = lax.rsqrt(hist + 2.0)
    dis2 = lax.rsqrt(hist + 1.0)
    srow = s0_ref[...] + s1_ref[...]
    pre = (srow * dis1[:, None]
           + (2.0 * dis1 * dis1)[:, None] * xw1_ref[...]
           + b1_ref[...])
    mu = jnp.mean(pre, axis=-1, keepdims=True)
    dvar = pre - mu
    var = jnp.mean(dvar * dvar, axis=-1, keepdims=True)
    t = dvar * lax.rsqrt(var + EPS) * g1_ref[...] + bb1_ref[...]
    h1 = jnp.where(t >= 0, t, 0.2 * t)
    xw2 = _dot(h1, w2_ref[...])
    xw2_ref[...] = xw2
    y2_ref[...] = xw2 * dis2[:, None]


def _tc3_body(s0_ref, s1_ref, xw2_ref, p0_ref, p1_ref, b2_ref, batch_ref,
              wa_ref, ba_ref, g2_ref, bb2_ref, wb_ref, bbb_ref,
              out_ref, pooled_ref):
    i = pl.program_id(0)

    @pl.when(i == 0)
    def _():
        pooled_ref[...] = jnp.full((G, H), -1e30, jnp.float32)

    hist = p0_ref[0, 0, :] + p1_ref[0, 0, :]
    dis2 = lax.rsqrt(hist + 1.0)
    srow = s0_ref[...] + s1_ref[...]
    t = (srow * dis2[:, None]
         + (dis2 * dis2)[:, None] * xw2_ref[...]
         + b2_ref[...])
    h2 = jnp.where(t >= 0, t, 0.2 * t)

    b2d = batch_ref[...]
    g_lo = b2d[0, 0]
    g_hi = b2d[_BR - 1, 0]

    def seg(g, _):
        vals = jnp.where(b2d == g, h2, -1e30)
        m = jnp.max(vals, axis=0)
        cur = pooled_ref[pl.ds(g, 1), :]
        pooled_ref[pl.ds(g, 1), :] = jnp.maximum(cur, m[None, :])
        return 0
    lax.fori_loop(g_lo, g_hi + 1, seg, 0)

    @pl.when(i == _NB - 1)
    def _():
        pooled = pooled_ref[...]
        z = _dot(pooled, wa_ref[...]) + ba_ref[...]
        mu = jnp.mean(z, axis=-1, keepdims=True)
        dv = z - mu
        var = jnp.mean(dv * dv, axis=-1, keepdims=True)
        z = dv * lax.rsqrt(var + EPS) * g2_ref[...] + bb2_ref[...]
        z = jnp.where(z >= 0, z, 0.2 * z)
        z2 = _dot(z, wb_ref[...]) + bbb_ref[...]
        colid = lax.broadcasted_iota(jnp.int32, (G, 128), 1)
        valid = colid < C
        zm = jnp.where(valid, z2, -1e30)
        m = jnp.max(zm, axis=-1, keepdims=True)
        e = jnp.where(valid, jnp.exp(zm - m), 0.0)
        out_ref[...] = e / jnp.sum(e, axis=-1, keepdims=True)


def _row_spec():
    return pl.BlockSpec((_BR, H), lambda i: (i, 0))


def _p_spec():
    return pl.BlockSpec((1, 1, _BR), lambda i: (i, 0, 0))


def _full2(shape):
    return pl.BlockSpec(shape, lambda i: (0,) * len(shape))


def kernel(x, edge_index, batch, W1, b1, ln1_g, ln1_b, W2, b2, Wa, ba,
           ln2_g, ln2_b, Wb, bb):
    row = edge_index[0]
    col = edge_index[1]

    hist2 = _hist_sc(col)
    p3 = hist2.reshape(2, _NB, _BR).transpose(1, 0, 2)
    p0 = p3[:, 0:1, :]
    p1 = p3[:, 1:2, :]

    y1, xw1 = pl.pallas_call(
        _tc1_body,
        grid=(_NB,),
        in_specs=[_row_spec(), _full2((D, H)), _p_spec(), _p_spec()],
        out_specs=[_row_spec(), _row_spec()],
        out_shape=[jax.ShapeDtypeStruct((N, H), jnp.float32)] * 2,
    )(x, W1, p0, p1)

    s1p = _spmm_sc(row, col, y1)

    b1r = b1.reshape(1, H)
    g1r = ln1_g.reshape(1, H)
    bb1r = ln1_b.reshape(1, H)
    y2, xw2 = pl.pallas_call(
        _tc2_body,
        grid=(_NB,),
        in_specs=[_row_spec(), _row_spec(), _row_spec(), _p_spec(), _p_spec(),
                  _full2((1, H)), _full2((1, H)), _full2((1, H)),
                  _full2((H, H))],
        out_specs=[_row_spec(), _row_spec()],
        out_shape=[jax.ShapeDtypeStruct((N, H), jnp.float32)] * 2,
    )(s1p[:N], s1p[N:], xw1, p0, p1, b1r, g1r, bb1r, W2)

    s2p = _spmm_sc(row, col, y2)

    batch3 = jnp.broadcast_to(batch[:, None], (N, 128))
    Wbp = jnp.pad(Wb, ((0, 0), (0, 128 - C)))
    bbp = jnp.pad(bb, (0, 128 - C)).reshape(1, 128)
    out = pl.pallas_call(
        _tc3_body,
        grid=(_NB,),
        in_specs=[_row_spec(), _row_spec(), _row_spec(), _p_spec(), _p_spec(),
                  _full2((1, H)),
                  pl.BlockSpec((_BR, 128), lambda i: (i, 0)),
                  _full2((H, 768)), _full2((1, 768)), _full2((1, 768)),
                  _full2((1, 768)), _full2((768, 128)), _full2((1, 128))],
        out_specs=pl.BlockSpec((G, 128), lambda i: (0, 0)),
        out_shape=jax.ShapeDtypeStruct((G, 128), jnp.float32),
        scratch_shapes=[pltpu.VMEM((G, H), jnp.float32)],
    )(s2p[:N], s2p[N:], xw2, p0, p1, b2.reshape(1, H), batch3,
      Wa, ba.reshape(1, 768), ln2_g.reshape(1, 768), ln2_b.reshape(1, 768),
      Wbp, bbp)
    return out[:, :C]


# speedup vs baseline: 42.4800x; 1.2761x over previous
import functools

import jax
import jax.numpy as jnp
from jax import lax
from jax.experimental import pallas as pl
from jax.experimental.pallas import tpu as pltpu
from jax.experimental.pallas import tpu_sc as plsc

N = 10000
E = 320000
D = 128
H = 128
C = 10
G = 64
EPS = 1e-5

NC = 2
NS = 16
NW = NC * NS
EPW = E // NW
CHUNK = 40
NSTEP = EPW // CHUNK
NBK = 10
NOUT = NSTEP // NBK
NB = 5
KD = 3

def _mesh():
    return plsc.VectorSubcoreMesh(core_axis_name="c", subcore_axis_name="s")


def _zero_vmem2d(ref, nrows):
    def body(i, _):
        for j in range(8):
            ref[i, pl.ds(j * 16, 16)] = jnp.zeros((16,), jnp.float32)
        return 0
    lax.fori_loop(0, nrows, body, 0)


_HNI = 8
_HNS = 4


def _hist_body(col_hbm, out_hbm, ones_v, zbuf_v, acc_sh, *refs):
    c = lax.axis_index("c")
    s = lax.axis_index("s")
    wid = s * NC + c
    cv = refs[:_HNI]
    isem = refs[_HNI:2 * _HNI]
    ssem = refs[2 * _HNI:2 * _HNI + _HNS]
    ebase = wid * EPW

    def idx_load(t, j):
        pltpu.async_copy(col_hbm.at[pl.ds(ebase + t * CHUNK, CHUNK)],
                         cv[j], isem[j])

    def idx_wait(j):
        pltpu.make_async_copy(col_hbm.at[pl.ds(0, CHUNK)], cv[j],
                              isem[j]).wait()

    def s_start(j, r):
        pltpu.async_copy(ones_v, acc_sh.at[cv[j]], ssem[r], add=True)

    def s_wait(r):
        pltpu.make_async_copy(ones_v, acc_sh.at[cv[0]], ssem[r]).wait()

    for j in range(CHUNK // 16):
        ones_v[pl.ds(j * 16, 16)] = jnp.ones((16,), jnp.float32)
    if CHUNK % 16:
        ones_v[pl.ds(CHUNK - 16, 16)] = jnp.ones((16,), jnp.float32)
    for t in range(_HNS):
        idx_load(t, t)
    def zb(i, _):
        zbuf_v[pl.ds(i * 16, 16)] = jnp.zeros((16,), jnp.float32)
        return 0
    lax.fori_loop(0, 62, zb, 0)
    zbuf_v[pl.ds(984, 16)] = jnp.zeros((16,), jnp.float32)

    @pl.when(s < 10)
    def _():
        pltpu.sync_copy(zbuf_v, acc_sh.at[pl.ds(s * 1000, 1000)])
    plsc.subcore_barrier()

    def outer(i, _):
        for b in range(_HNI):
            t = i * _HNI + b

            @pl.when(t >= _HNS)
            def _():
                s_wait(b % _HNS)

            @pl.when(t + _HNS < NSTEP)
            def _():
                idx_load(t + _HNS, (b + _HNS) % _HNI)

            idx_wait(b)
            s_start(b, b % _HNS)
        return 0
    lax.fori_loop(0, NSTEP // _HNI, outer, 0)
    for t in range(NSTEP - NSTEP % _HNI, NSTEP):
        b = t % _HNI

        @pl.when(t >= _HNS)
        def _():
            s_wait(b % _HNS)

        idx_wait(b)
        s_start(b, b % _HNS)
    for r in range(_HNS):
        s_wait(r)
    plsc.subcore_barrier()

    @pl.when(s < 10)
    def _():
        pltpu.sync_copy(acc_sh.at[pl.ds(s * 1000, 1000)], zbuf_v)
        pltpu.sync_copy(zbuf_v, out_hbm.at[pl.ds(c * N + s * 1000, 1000)])


def _hist_sc(col):
    k = pl.kernel(
        _hist_body,
        mesh=_mesh(),
        out_type=jax.ShapeDtypeStruct((2 * N,), jnp.float32),
        scratch_types=[
            pltpu.VMEM((CHUNK,), jnp.float32),
            pltpu.VMEM((1000,), jnp.float32),
            pltpu.VMEM_SHARED((N,), jnp.float32),
        ] + [pltpu.VMEM((CHUNK,), jnp.int32)] * _HNI
          + [pltpu.SemaphoreType.DMA] * (_HNI + _HNS),
    )
    return k(col)


NI = 10


def _spmm_body(row_hbm, col_hbm, y_hbm, out_hbm, bufs, acc_sh, *refs):
    c = lax.axis_index("c")
    s = lax.axis_index("s")
    wid = s * NC + c
    rv = refs[:NI]
    cv = refs[NI:2 * NI]
    isem = refs[2 * NI:3 * NI]
    gsem = refs[3 * NI:3 * NI + NB]
    ssem = refs[3 * NI + NB:3 * NI + 2 * NB]
    ebase = wid * EPW

    def bufref(b):
        return bufs.at[pl.ds(b * CHUNK, CHUNK)]

    def idx_load(t, j):
        pltpu.async_copy(row_hbm.at[pl.ds(ebase + t * CHUNK, CHUNK)],
                         rv[j], isem[j])
        pltpu.async_copy(col_hbm.at[pl.ds(ebase + t * CHUNK, CHUNK)],
                         cv[j], isem[j])

    def idx_wait(j):
        pltpu.make_async_copy(row_hbm.at[pl.ds(0, CHUNK)], rv[j],
                              isem[j]).wait()
        pltpu.make_async_copy(col_hbm.at[pl.ds(0, CHUNK)], cv[j],
                              isem[j]).wait()

    def g_start(j, b):
        pltpu.async_copy(y_hbm.at[rv[j]], bufref(b), gsem[b])

    def g_wait(b):
        pltpu.make_async_copy(y_hbm.at[rv[0]], bufref(b), gsem[b]).wait()

    def s_start(j, b):
        pltpu.async_copy(bufref(b), acc_sh.at[cv[j]], ssem[b], add=True)

    def s_wait(b):
        pltpu.make_async_copy(bufref(b), acc_sh.at[cv[0]], ssem[b]).wait()

    for t in range(NI - 3):
        idx_load(t, t)
    _zero_vmem2d(bufs, NB * CHUNK)
    @pl.when(s < 10)
    def _():
        for k in range(5):
            pltpu.sync_copy(bufs, acc_sh.at[pl.ds(s * 1000 + k * 200, 200)])
    plsc.subcore_barrier()

    for t in range(KD):
        idx_wait(t)
        g_start(t, t)

    def outer(i, _):
        for b in range(NBK):
            t = i * NBK + b
            bb = b % NB
            bg = (b + KD) % NB

            @pl.when(t + NI - 3 < NSTEP)
            def _():
                idx_load(t + NI - 3, (b + NI - 3) % NI)

            @pl.when(t >= NB - KD)
            def _():
                s_wait(bg)

            @pl.when(t + KD < NSTEP)
            def _():
                idx_wait((b + KD) % NI)
                g_start((b + KD) % NI, bg)

            g_wait(bb)
            s_start(b, bb)
        return 0
    lax.fori_loop(0, NOUT, outer, 0)

    for t in range(NSTEP - (NB - KD), NSTEP):
        s_wait(t % NB)
    plsc.subcore_barrier()

    @pl.when(s < 10)
    def _():
        for k in range(5):
            r0 = s * 1000 + k * 200
            pltpu.sync_copy(acc_sh.at[pl.ds(r0, 200)], bufs)
            pltpu.sync_copy(bufs, out_hbm.at[pl.ds(c * N + r0, 200)])


def _spmm_sc(row, col, y):
    k = pl.kernel(
        _spmm_body,
        mesh=_mesh(),
        out_type=jax.ShapeDtypeStruct((2 * N, H), jnp.float32),
        scratch_types=[
            pltpu.VMEM((NB * CHUNK, H), jnp.float32),
            pltpu.VMEM_SHARED((N, H), jnp.float32),
        ] + [pltpu.VMEM((CHUNK,), jnp.int32)] * (2 * NI)
          + [pltpu.SemaphoreType.DMA] * (NI + 2 * NB),
    )
    return k(row, col, y)


_BR = 1000
_NB = N // _BR


def _dot(a, b):
    return jnp.dot(a, b, preferred_element_type=jnp.float32,
                   precision=lax.Precision.HIGHEST)


def _tc1_body(x_ref, w1_ref, p0_ref, p1_ref, y1_ref, xw1_ref):
    xw = _dot(x_ref[...], w1_ref[...])
    hist = p0_ref[0, 0, :] + p1_ref[0, 0, :]
    dis = lax.rsqrt(hist + 2.0)
    xw1_ref[...] = xw
    y1_ref[...] = xw * dis[:, None]


def _tc2_body(s0_ref, s1_ref, xw1_ref, p0_ref, p1_ref, b1_ref, g1_ref,
              bb1_ref, w2_ref, y2_ref, xw2_ref):
    hist = p0_ref[0, 0, :] + p1_ref[0, 0, :]
    dis1 = lax.rsqrt(hist + 2.0)
    dis2 = lax.rsqrt(hist + 1.0)
    srow = s0_ref[...] + s1_ref[...]
    pre = (srow * dis1[:, None]
           + (2.0 * dis1 * dis1)[:, None] * xw1_ref[...]
           + b1_ref[...])
    mu = jnp.mean(pre, axis=-1, keepdims=True)
    dvar = pre - mu
    var = jnp.mean(dvar * dvar, axis=-1, keepdims=True)
    t = dvar * lax.rsqrt(var + EPS) * g1_ref[...] + bb1_ref[...]
    h1 = jnp.where(t >= 0, t, 0.2 * t)
    xw2 = _dot(h1, w2_ref[...])
    xw2_ref[...] = xw2
    y2_ref[...] = xw2 * dis2[:, None]


def _tc3_body(s0_ref, s1_ref, xw2_ref, p0_ref, p1_ref, b2_ref, batch_ref,
              wa_ref, ba_ref, g2_ref, bb2_ref, wb_ref, bbb_ref,
              out_ref, pooled_ref):
    i = pl.program_id(0)

    @pl.when(i == 0)
    def _():
        pooled_ref[...] = jnp.full((G, H), -1e30, jnp.float32)

    hist = p0_ref[0, 0, :] + p1_ref[0, 0, :]
    dis2 = lax.rsqrt(hist + 1.0)
    srow = s0_ref[...] + s1_ref[...]
    t = (srow * dis2[:, None]
         + (dis2 * dis2)[:, None] * xw2_ref[...]
         + b2_ref[...])
    h2 = jnp.where(t >= 0, t, 0.2 * t)

    b2d = batch_ref[...][:, 0:1]
    g_lo = b2d[0, 0]
    g_hi = b2d[_BR - 1, 0]

    def seg(g, _):
        vals = jnp.where(b2d == g, h2, -1e30)
        m = jnp.max(vals, axis=0)
        cur = pooled_ref[pl.ds(g, 1), :]
        pooled_ref[pl.ds(g, 1), :] = jnp.maximum(cur, m[None, :])
        return 0
    lax.fori_loop(g_lo, g_hi + 1, seg, 0)

    @pl.when(i == _NB - 1)
    def _():
        pooled = pooled_ref[...]
        z = _dot(pooled, wa_ref[...]) + ba_ref[...]
        mu = jnp.mean(z, axis=-1, keepdims=True)
        dv = z - mu
        var = jnp.mean(dv * dv, axis=-1, keepdims=True)
        z = dv * lax.rsqrt(var + EPS) * g2_ref[...] + bb2_ref[...]
        z = jnp.where(z >= 0, z, 0.2 * z)
        z2 = _dot(z, wb_ref[...]) + bbb_ref[...]
        colid = lax.broadcasted_iota(jnp.int32, (G, 128), 1)
        valid = colid < C
        zm = jnp.where(valid, z2, -1e30)
        m = jnp.max(zm, axis=-1, keepdims=True)
        e = jnp.where(valid, jnp.exp(zm - m), 0.0)
        out_ref[...] = e / jnp.sum(e, axis=-1, keepdims=True)


def _row_spec():
    return pl.BlockSpec((_BR, H), lambda i: (i, 0))


def _p_spec():
    return pl.BlockSpec((1, 1, _BR), lambda i: (i, 0, 0))


def _full2(shape):
    return pl.BlockSpec(shape, lambda i: (0,) * len(shape))


def kernel(x, edge_index, batch, W1, b1, ln1_g, ln1_b, W2, b2, Wa, ba,
           ln2_g, ln2_b, Wb, bb):
    row = edge_index[0]
    col = edge_index[1]

    hist2 = _hist_sc(col)
    p3 = hist2.reshape(2, _NB, _BR).transpose(1, 0, 2)
    p0 = p3[:, 0:1, :]
    p1 = p3[:, 1:2, :]

    y1, xw1 = pl.pallas_call(
        _tc1_body,
        grid=(_NB,),
        in_specs=[_row_spec(), _full2((D, H)), _p_spec(), _p_spec()],
        out_specs=[_row_spec(), _row_spec()],
        out_shape=[jax.ShapeDtypeStruct((N, H), jnp.float32)] * 2,
    )(x, W1, p0, p1)

    s1p = _spmm_sc(row, col, y1)

    b1r = b1.reshape(1, H)
    g1r = ln1_g.reshape(1, H)
    bb1r = ln1_b.reshape(1, H)
    y2, xw2 = pl.pallas_call(
        _tc2_body,
        grid=(_NB,),
        in_specs=[_row_spec(), _row_spec(), _row_spec(), _p_spec(), _p_spec(),
                  _full2((1, H)), _full2((1, H)), _full2((1, H)),
                  _full2((H, H))],
        out_specs=[_row_spec(), _row_spec()],
        out_shape=[jax.ShapeDtypeStruct((N, H), jnp.float32)] * 2,
    )(s1p[:N], s1p[N:], xw1, p0, p1, b1r, g1r, bb1r, W2)

    s2p = _spmm_sc(row, col, y2)

    batch3 = jnp.broadcast_to(batch[:, None], (N, 8))
    Wbp = jnp.pad(Wb, ((0, 0), (0, 128 - C)))
    bbp = jnp.pad(bb, (0, 128 - C)).reshape(1, 128)
    out = pl.pallas_call(
        _tc3_body,
        grid=(_NB,),
        in_specs=[_row_spec(), _row_spec(), _row_spec(), _p_spec(), _p_spec(),
                  _full2((1, H)),
                  pl.BlockSpec((_BR, 8), lambda i: (i, 0)),
                  _full2((H, 768)), _full2((1, 768)), _full2((1, 768)),
                  _full2((1, 768)), _full2((768, 128)), _full2((1, 128))],
        out_specs=pl.BlockSpec((G, 128), lambda i: (0, 0)),
        out_shape=jax.ShapeDtypeStruct((G, 128), jnp.float32),
        scratch_shapes=[pltpu.VMEM((G, H), jnp.float32)],
    )(s2p[:N], s2p[N:], xw2, p0, p1, b2.reshape(1, H), batch3,
      Wa, ba.reshape(1, 768), ln2_g.reshape(1, 768), ln2_b.reshape(1, 768),
      Wbp, bbp)
    return out[:, :C]


# speedup vs baseline: 44.5655x; 1.0491x over previous
import functools

import jax
import jax.numpy as jnp
from jax import lax
from jax.experimental import pallas as pl
from jax.experimental.pallas import tpu as pltpu
from jax.experimental.pallas import tpu_sc as plsc

N = 10000
E = 320000
D = 128
H = 128
C = 10
G = 64
EPS = 1e-5

NC = 2
NS = 16
NW = NC * NS
EPW = E // NW
CHUNK = 40
NSTEP = EPW // CHUNK
NBK = 10
NOUT = NSTEP // NBK
NB = 5
KD = 3

def _mesh():
    return plsc.VectorSubcoreMesh(core_axis_name="c", subcore_axis_name="s")


def _zero_vmem2d(ref, nrows):
    def body(i, _):
        for j in range(8):
            ref[i, pl.ds(j * 16, 16)] = jnp.zeros((16,), jnp.float32)
        return 0
    lax.fori_loop(0, nrows, body, 0)


_HNI = 8
_HNS = 4


def _hist_body(col_hbm, out_hbm, ones_v, zbuf_v, acc_sh, *refs):
    c = lax.axis_index("c")
    s = lax.axis_index("s")
    wid = s * NC + c
    cv = refs[:_HNI]
    isem = refs[_HNI:2 * _HNI]
    ssem = refs[2 * _HNI:2 * _HNI + _HNS]
    ebase = wid * EPW

    def idx_load(t, j):
        pltpu.async_copy(col_hbm.at[pl.ds(ebase + t * CHUNK, CHUNK)],
                         cv[j], isem[j])

    def idx_wait(j):
        pltpu.make_async_copy(col_hbm.at[pl.ds(0, CHUNK)], cv[j],
                              isem[j]).wait()

    def s_start(j, r):
        pltpu.async_copy(ones_v, acc_sh.at[cv[j]], ssem[r], add=True)

    def s_wait(r):
        pltpu.make_async_copy(ones_v, acc_sh.at[cv[0]], ssem[r]).wait()

    for j in range(CHUNK // 16):
        ones_v[pl.ds(j * 16, 16)] = jnp.ones((16,), jnp.float32)
    if CHUNK % 16:
        ones_v[pl.ds(CHUNK - 16, 16)] = jnp.ones((16,), jnp.float32)
    for t in range(_HNS):
        idx_load(t, t)
    def zb(i, _):
        zbuf_v[pl.ds(i * 16, 16)] = jnp.zeros((16,), jnp.float32)
        return 0
    lax.fori_loop(0, 62, zb, 0)
    zbuf_v[pl.ds(984, 16)] = jnp.zeros((16,), jnp.float32)

    @pl.when(s < 10)
    def _():
        pltpu.sync_copy(zbuf_v, acc_sh.at[pl.ds(s * 1000, 1000)])
    plsc.subcore_barrier()

    def outer(i, _):
        for b in range(_HNI):
            t = i * _HNI + b

            @pl.when(t >= _HNS)
            def _():
                s_wait(b % _HNS)

            @pl.when(t + _HNS < NSTEP)
            def _():
                idx_load(t + _HNS, (b + _HNS) % _HNI)

            idx_wait(b)
            s_start(b, b % _HNS)
        return 0
    lax.fori_loop(0, NSTEP // _HNI, outer, 0)
    for t in range(NSTEP - NSTEP % _HNI, NSTEP):
        b = t % _HNI

        @pl.when(t >= _HNS)
        def _():
            s_wait(b % _HNS)

        idx_wait(b)
        s_start(b, b % _HNS)
    for r in range(_HNS):
        s_wait(r)
    plsc.subcore_barrier()

    @pl.when(s < 10)
    def _():
        pltpu.sync_copy(acc_sh.at[pl.ds(s * 1000, 1000)], zbuf_v)
        pltpu.sync_copy(zbuf_v, out_hbm.at[pl.ds(s * 2000 + c * 1000, 1000)])


def _hist_sc(col):
    k = pl.kernel(
        _hist_body,
        mesh=_mesh(),
        out_type=jax.ShapeDtypeStruct((2 * N,), jnp.float32),
        scratch_types=[
            pltpu.VMEM((CHUNK,), jnp.float32),
            pltpu.VMEM((1000,), jnp.float32),
            pltpu.VMEM_SHARED((N,), jnp.float32),
        ] + [pltpu.VMEM((CHUNK,), jnp.int32)] * _HNI
          + [pltpu.SemaphoreType.DMA] * (_HNI + _HNS),
    )
    return k(col)


NI = 10


def _spmm_body(row_hbm, col_hbm, y_hbm, out_hbm, bufs, acc_sh, *refs):
    c = lax.axis_index("c")
    s = lax.axis_index("s")
    wid = s * NC + c
    rv = refs[:NI]
    cv = refs[NI:2 * NI]
    isem = refs[2 * NI:3 * NI]
    gsem = refs[3 * NI:3 * NI + NB]
    ssem = refs[3 * NI + NB:3 * NI + 2 * NB]
    ebase = wid * EPW

    def bufref(b):
        return bufs.at[pl.ds(b * CHUNK, CHUNK)]

    def idx_load(t, j):
        pltpu.async_copy(row_hbm.at[pl.ds(ebase + t * CHUNK, CHUNK)],
                         rv[j], isem[j])
        pltpu.async_copy(col_hbm.at[pl.ds(ebase + t * CHUNK, CHUNK)],
                         cv[j], isem[j])

    def idx_wait(j):
        pltpu.make_async_copy(row_hbm.at[pl.ds(0, CHUNK)], rv[j],
                              isem[j]).wait()
        pltpu.make_async_copy(col_hbm.at[pl.ds(0, CHUNK)], cv[j],
                              isem[j]).wait()

    def g_start(j, b):
        pltpu.async_copy(y_hbm.at[rv[j]], bufref(b), gsem[b])

    def g_wait(b):
        pltpu.make_async_copy(y_hbm.at[rv[0]], bufref(b), gsem[b]).wait()

    def s_start(j, b):
        pltpu.async_copy(bufref(b), acc_sh.at[cv[j]], ssem[b], add=True)

    def s_wait(b):
        pltpu.make_async_copy(bufref(b), acc_sh.at[cv[0]], ssem[b]).wait()

    for t in range(NI - 3):
        idx_load(t, t)
    _zero_vmem2d(bufs, NB * CHUNK)
    @pl.when(s < 10)
    def _():
        for k in range(5):
            pltpu.sync_copy(bufs, acc_sh.at[pl.ds(s * 1000 + k * 200, 200)])
    plsc.subcore_barrier()

    for t in range(KD):
        idx_wait(t)
        g_start(t, t)

    def outer(i, _):
        for b in range(NBK):
            t = i * NBK + b
            bb = b % NB
            bg = (b + KD) % NB

            @pl.when(t + NI - 3 < NSTEP)
            def _():
                idx_load(t + NI - 3, (b + NI - 3) % NI)

            @pl.when(t >= NB - KD)
            def _():
                s_wait(bg)

            @pl.when(t + KD < NSTEP)
            def _():
                idx_wait((b + KD) % NI)
                g_start((b + KD) % NI, bg)

            g_wait(bb)
            s_start(b, bb)
        return 0
    lax.fori_loop(0, NOUT, outer, 0)

    for t in range(NSTEP - (NB - KD), NSTEP):
        s_wait(t % NB)
    plsc.subcore_barrier()

    @pl.when(s < 10)
    def _():
        for k in range(5):
            r0 = s * 1000 + k * 200
            pltpu.sync_copy(acc_sh.at[pl.ds(r0, 200)], bufs)
            pltpu.sync_copy(bufs, out_hbm.at[pl.ds(c * N + r0, 200)])


def _spmm_sc(row, col, y):
    k = pl.kernel(
        _spmm_body,
        mesh=_mesh(),
        out_type=jax.ShapeDtypeStruct((2 * N, H), jnp.float32),
        scratch_types=[
            pltpu.VMEM((NB * CHUNK, H), jnp.float32),
            pltpu.VMEM_SHARED((N, H), jnp.float32),
        ] + [pltpu.VMEM((CHUNK,), jnp.int32)] * (2 * NI)
          + [pltpu.SemaphoreType.DMA] * (NI + 2 * NB),
    )
    return k(row, col, y)


_BR = 1000
_NB = N // _BR


def _dot(a, b):
    return jnp.dot(a, b, preferred_element_type=jnp.float32,
                   precision=lax.Precision.HIGHEST)


def _tc1_body(x_ref, w1_ref, p_ref, y1_ref):
    xw = _dot(x_ref[...], w1_ref[...])
    hist = p_ref[0, 0, :] + p_ref[0, 1, :]
    dis = lax.rsqrt(hist + 2.0)
    y1_ref[...] = xw * dis[:, None]


def _tc2_body(s0_ref, s1_ref, y1_ref, p_ref, b1_ref, g1_ref,
              bb1_ref, w2_ref, y2_ref):
    hist = p_ref[0, 0, :] + p_ref[0, 1, :]
    dis1 = lax.rsqrt(hist + 2.0)
    dis2 = lax.rsqrt(hist + 1.0)
    srow = s0_ref[...] + s1_ref[...]
    pre = (srow * dis1[:, None]
           + (2.0 * dis1)[:, None] * y1_ref[...]
           + b1_ref[...])
    mu = jnp.mean(pre, axis=-1, keepdims=True)
    dvar = pre - mu
    var = jnp.mean(dvar * dvar, axis=-1, keepdims=True)
    t = dvar * lax.rsqrt(var + EPS) * g1_ref[...] + bb1_ref[...]
    h1 = jnp.where(t >= 0, t, 0.2 * t)
    xw2 = _dot(h1, w2_ref[...])
    y2_ref[...] = xw2 * dis2[:, None]


def _tc3_body(s0_ref, s1_ref, y2_ref, p_ref, b2_ref, batch_ref,
              wa_ref, ba_ref, g2_ref, bb2_ref, wb_ref, bbb_ref,
              out_ref, pooled_ref):
    i = pl.program_id(0)

    @pl.when(i == 0)
    def _():
        pooled_ref[...] = jnp.full((G, H), -1e30, jnp.float32)

    hist = p_ref[0, 0, :] + p_ref[0, 1, :]
    dis2 = lax.rsqrt(hist + 1.0)
    srow = s0_ref[...] + s1_ref[...]
    t = (srow * dis2[:, None]
         + dis2[:, None] * y2_ref[...]
         + b2_ref[...])
    h2 = jnp.where(t >= 0, t, 0.2 * t)

    b2d = batch_ref[...][:, 0:1]
    g_lo = b2d[0, 0]
    g_hi = b2d[_BR - 1, 0]

    def seg(g, _):
        vals = jnp.where(b2d == g, h2, -1e30)
        m = jnp.max(vals, axis=0)
        cur = pooled_ref[pl.ds(g, 1), :]
        pooled_ref[pl.ds(g, 1), :] = jnp.maximum(cur, m[None, :])
        return 0
    lax.fori_loop(g_lo, g_hi + 1, seg, 0)

    @pl.when(i == _NB - 1)
    def _():
        pooled = pooled_ref[...]
        z = _dot(pooled, wa_ref[...]) + ba_ref[...]
        mu = jnp.mean(z, axis=-1, keepdims=True)
        dv = z - mu
        var = jnp.mean(dv * dv, axis=-1, keepdims=True)
        z = dv * lax.rsqrt(var + EPS) * g2_ref[...] + bb2_ref[...]
        z = jnp.where(z >= 0, z, 0.2 * z)
        z2 = _dot(z, wb_ref[...]) + bbb_ref[...]
        colid = lax.broadcasted_iota(jnp.int32, (G, 128), 1)
        valid = colid < C
        zm = jnp.where(valid, z2, -1e30)
        m = jnp.max(zm, axis=-1, keepdims=True)
        e = jnp.where(valid, jnp.exp(zm - m), 0.0)
        out_ref[...] = e / jnp.sum(e, axis=-1, keepdims=True)


def _row_spec():
    return pl.BlockSpec((_BR, H), lambda i: (i, 0))


def _row_spec_hi():
    return pl.BlockSpec((_BR, H), lambda i: (i + _NB, 0))


def _p_spec():
    return pl.BlockSpec((1, 2, _BR), lambda i: (i, 0, 0))


def _full2(shape):
    return pl.BlockSpec(shape, lambda i: (0,) * len(shape))


def kernel(x, edge_index, batch, W1, b1, ln1_g, ln1_b, W2, b2, Wa, ba,
           ln2_g, ln2_b, Wb, bb):
    row = edge_index[0]
    col = edge_index[1]

    hist2 = _hist_sc(col)
    p3 = hist2.reshape(_NB, 2, _BR)

    y1 = pl.pallas_call(
        _tc1_body,
        grid=(_NB,),
        in_specs=[_row_spec(), _full2((D, H)), _p_spec()],
        out_specs=_row_spec(),
        out_shape=jax.ShapeDtypeStruct((N, H), jnp.float32),
    )(x, W1, p3)

    s1p = _spmm_sc(row, col, y1)

    b1r = b1.reshape(1, H)
    g1r = ln1_g.reshape(1, H)
    bb1r = ln1_b.reshape(1, H)
    y2 = pl.pallas_call(
        _tc2_body,
        grid=(_NB,),
        in_specs=[_row_spec(), _row_spec_hi(), _row_spec(), _p_spec(),
                  _full2((1, H)), _full2((1, H)), _full2((1, H)),
                  _full2((H, H))],
        out_specs=_row_spec(),
        out_shape=jax.ShapeDtypeStruct((N, H), jnp.float32),
    )(s1p, s1p, y1, p3, b1r, g1r, bb1r, W2)

    s2p = _spmm_sc(row, col, y2)

    batch3 = jnp.broadcast_to(batch[:, None], (N, 8))
    Wbp = jnp.pad(Wb, ((0, 0), (0, 128 - C)))
    bbp = jnp.pad(bb, (0, 128 - C)).reshape(1, 128)
    out = pl.pallas_call(
        _tc3_body,
        grid=(_NB,),
        in_specs=[_row_spec(), _row_spec_hi(), _row_spec(), _p_spec(),
                  _full2((1, H)),
                  pl.BlockSpec((_BR, 8), lambda i: (i, 0)),
                  _full2((H, 768)), _full2((1, 768)), _full2((1, 768)),
                  _full2((1, 768)), _full2((768, 128)), _full2((1, 128))],
        out_specs=pl.BlockSpec((G, 128), lambda i: (0, 0)),
        out_shape=jax.ShapeDtypeStruct((G, 128), jnp.float32),
        scratch_shapes=[pltpu.VMEM((G, H), jnp.float32)],
    )(s2p, s2p, y2, p3, b2.reshape(1, H), batch3,
      Wa, ba.reshape(1, 768), ln2_g.reshape(1, 768), ln2_b.reshape(1, 768),
      Wbp, bbp)
    return out[:, :C]


# speedup vs baseline: 45.8629x; 1.0291x over previous
import functools

import jax
import jax.numpy as jnp
from jax import lax
from jax.experimental import pallas as pl
from jax.experimental.pallas import tpu as pltpu
from jax.experimental.pallas import tpu_sc as plsc

N = 10000
E = 320000
D = 128
H = 128
C = 10
G = 64
EPS = 1e-5

NC = 2
NS = 16
NW = NC * NS
EPW = E // NW
CHUNK = 40
NSTEP = EPW // CHUNK
NBK = 10
NOUT = NSTEP // NBK
NB = 5
KD = 4

def _mesh():
    return plsc.VectorSubcoreMesh(core_axis_name="c", subcore_axis_name="s")


def _zero_vmem2d(ref, nrows):
    def body(i, _):
        for j in range(8):
            ref[i, pl.ds(j * 16, 16)] = jnp.zeros((16,), jnp.float32)
        return 0
    lax.fori_loop(0, nrows, body, 0)


_HNI = 8
_HNS = 4


def _hist_body(col_hbm, out_hbm, ones_v, zbuf_v, acc_sh, *refs):
    c = lax.axis_index("c")
    s = lax.axis_index("s")
    wid = s * NC + c
    cv = refs[:_HNI]
    isem = refs[_HNI:2 * _HNI]
    ssem = refs[2 * _HNI:2 * _HNI + _HNS]
    ebase = wid * EPW

    def idx_load(t, j):
        pltpu.async_copy(col_hbm.at[pl.ds(ebase + t * CHUNK, CHUNK)],
                         cv[j], isem[j])

    def idx_wait(j):
        pltpu.make_async_copy(col_hbm.at[pl.ds(0, CHUNK)], cv[j],
                              isem[j]).wait()

    def s_start(j, r):
        pltpu.async_copy(ones_v, acc_sh.at[cv[j]], ssem[r], add=True)

    def s_wait(r):
        pltpu.make_async_copy(ones_v, acc_sh.at[cv[0]], ssem[r]).wait()

    for j in range(CHUNK // 16):
        ones_v[pl.ds(j * 16, 16)] = jnp.ones((16,), jnp.float32)
    if CHUNK % 16:
        ones_v[pl.ds(CHUNK - 16, 16)] = jnp.ones((16,), jnp.float32)
    for t in range(_HNS):
        idx_load(t, t)
    def zb(i, _):
        zbuf_v[pl.ds(i * 16, 16)] = jnp.zeros((16,), jnp.float32)
        return 0
    lax.fori_loop(0, 62, zb, 0)
    zbuf_v[pl.ds(984, 16)] = jnp.zeros((16,), jnp.float32)

    @pl.when(s < 10)
    def _():
        pltpu.sync_copy(zbuf_v, acc_sh.at[pl.ds(s * 1000, 1000)])
    plsc.subcore_barrier()

    def outer(i, _):
        for b in range(_HNI):
            t = i * _HNI + b

            @pl.when(t >= _HNS)
            def _():
                s_wait(b % _HNS)

            @pl.when(t + _HNS < NSTEP)
            def _():
                idx_load(t + _HNS, (b + _HNS) % _HNI)

            idx_wait(b)
            s_start(b, b % _HNS)
        return 0
    lax.fori_loop(0, NSTEP // _HNI, outer, 0)
    for t in range(NSTEP - NSTEP % _HNI, NSTEP):
        b = t % _HNI

        @pl.when(t >= _HNS)
        def _():
            s_wait(b % _HNS)

        idx_wait(b)
        s_start(b, b % _HNS)
    for r in range(_HNS):
        s_wait(r)
    plsc.subcore_barrier()

    @pl.when(s < 10)
    def _():
        pltpu.sync_copy(acc_sh.at[pl.ds(s * 1000, 1000)], zbuf_v)
        pltpu.sync_copy(zbuf_v, out_hbm.at[pl.ds(s * 2000 + c * 1000, 1000)])


def _hist_sc(col):
    k = pl.kernel(
        _hist_body,
        mesh=_mesh(),
        out_type=jax.ShapeDtypeStruct((2 * N,), jnp.float32),
        scratch_types=[
            pltpu.VMEM((CHUNK,), jnp.float32),
            pltpu.VMEM((1000,), jnp.float32),
            pltpu.VMEM_SHARED((N,), jnp.float32),
        ] + [pltpu.VMEM((CHUNK,), jnp.int32)] * _HNI
          + [pltpu.SemaphoreType.DMA] * (_HNI + _HNS),
    )
    return k(col)


NI = 10


def _spmm_body(row_hbm, col_hbm, y_hbm, out_hbm, bufs, acc_sh, *refs):
    c = lax.axis_index("c")
    s = lax.axis_index("s")
    wid = s * NC + c
    rv = refs[:NI]
    cv = refs[NI:2 * NI]
    isem = refs[2 * NI:3 * NI]
    gsem = refs[3 * NI:3 * NI + NB]
    ssem = refs[3 * NI + NB:3 * NI + 2 * NB]
    ebase = wid * EPW

    def bufref(b):
        return bufs.at[pl.ds(b * CHUNK, CHUNK)]

    def idx_load(t, j):
        pltpu.async_copy(row_hbm.at[pl.ds(ebase + t * CHUNK, CHUNK)],
                         rv[j], isem[j])
        pltpu.async_copy(col_hbm.at[pl.ds(ebase + t * CHUNK, CHUNK)],
                         cv[j], isem[j])

    def idx_wait(j):
        pltpu.make_async_copy(row_hbm.at[pl.ds(0, CHUNK)], rv[j],
                              isem[j]).wait()
        pltpu.make_async_copy(col_hbm.at[pl.ds(0, CHUNK)], cv[j],
                              isem[j]).wait()

    def g_start(j, b):
        pltpu.async_copy(y_hbm.at[rv[j]], bufref(b), gsem[b])

    def g_wait(b):
        pltpu.make_async_copy(y_hbm.at[rv[0]], bufref(b), gsem[b]).wait()

    def s_start(j, b):
        pltpu.async_copy(bufref(b), acc_sh.at[cv[j]], ssem[b], add=True)

    def s_wait(b):
        pltpu.make_async_copy(bufref(b), acc_sh.at[cv[0]], ssem[b]).wait()

    for t in range(NI - 3):
        idx_load(t, t)
    _zero_vmem2d(bufs, NB * CHUNK)
    @pl.when(s < 10)
    def _():
        for k in range(5):
            pltpu.sync_copy(bufs, acc_sh.at[pl.ds(s * 1000 + k * 200, 200)])
    plsc.subcore_barrier()

    for t in range(KD):
        idx_wait(t)
        g_start(t, t)

    def outer(i, _):
        for b in range(NBK):
            t = i * NBK + b
            bb = b % NB
            bg = (b + KD) % NB

            @pl.when(t + NI - 3 < NSTEP)
            def _():
                idx_load(t + NI - 3, (b + NI - 3) % NI)

            @pl.when(t >= NB - KD)
            def _():
                s_wait(bg)

            @pl.when(t + KD < NSTEP)
            def _():
                idx_wait((b + KD) % NI)
                g_start((b + KD) % NI, bg)

            g_wait(bb)
            s_start(b, bb)
        return 0
    lax.fori_loop(0, NOUT, outer, 0)

    for t in range(NSTEP - (NB - KD), NSTEP):
        s_wait(t % NB)
    plsc.subcore_barrier()

    @pl.when(s < 10)
    def _():
        for k in range(5):
            r0 = s * 1000 + k * 200
            pltpu.sync_copy(acc_sh.at[pl.ds(r0, 200)], bufs)
            pltpu.sync_copy(bufs, out_hbm.at[pl.ds(c * N + r0, 200)])


def _spmm_sc(row, col, y):
    k = pl.kernel(
        _spmm_body,
        mesh=_mesh(),
        out_type=jax.ShapeDtypeStruct((2 * N, H), jnp.float32),
        scratch_types=[
            pltpu.VMEM((NB * CHUNK, H), jnp.float32),
            pltpu.VMEM_SHARED((N, H), jnp.float32),
        ] + [pltpu.VMEM((CHUNK,), jnp.int32)] * (2 * NI)
          + [pltpu.SemaphoreType.DMA] * (NI + 2 * NB),
    )
    return k(row, col, y)


_BR = 1000
_NB = N // _BR


def _dot(a, b):
    return jnp.dot(a, b, preferred_element_type=jnp.float32,
                   precision=lax.Precision.HIGHEST)


def _tc1_body(x_ref, w1_ref, p_ref, y1_ref):
    xw = _dot(x_ref[...], w1_ref[...])
    hist = p_ref[0, 0, :] + p_ref[0, 1, :]
    dis = lax.rsqrt(hist + 2.0)
    y1_ref[...] = xw * dis[:, None]


def _tc2_body(s0_ref, s1_ref, y1_ref, p_ref, b1_ref, g1_ref,
              bb1_ref, w2_ref, y2_ref):
    hist = p_ref[0, 0, :] + p_ref[0, 1, :]
    dis1 = lax.rsqrt(hist + 2.0)
    dis2 = lax.rsqrt(hist + 1.0)
    srow = s0_ref[...] + s1_ref[...]
    pre = (srow * dis1[:, None]
           + (2.0 * dis1)[:, None] * y1_ref[...]
           + b1_ref[...])
    mu = jnp.mean(pre, axis=-1, keepdims=True)
    dvar = pre - mu
    var = jnp.mean(dvar * dvar, axis=-1, keepdims=True)
    t = dvar * lax.rsqrt(var + EPS) * g1_ref[...] + bb1_ref[...]
    h1 = jnp.where(t >= 0, t, 0.2 * t)
    xw2 = _dot(h1, w2_ref[...])
    y2_ref[...] = xw2 * dis2[:, None]


def _tc3_body(s0_ref, s1_ref, y2_ref, p_ref, b2_ref, batch_ref,
              wa_ref, ba_ref, g2_ref, bb2_ref, wb_ref, bbb_ref,
              out_ref, pooled_ref):
    i = pl.program_id(0)

    @pl.when(i == 0)
    def _():
        pooled_ref[...] = jnp.full((G, H), -1e30, jnp.float32)

    hist = p_ref[0, 0, :] + p_ref[0, 1, :]
    dis2 = lax.rsqrt(hist + 1.0)
    srow = s0_ref[...] + s1_ref[...]
    t = (srow * dis2[:, None]
         + dis2[:, None] * y2_ref[...]
         + b2_ref[...])
    h2 = jnp.where(t >= 0, t, 0.2 * t)

    b2d = batch_ref[...][:, 0:1]
    g_lo = b2d[0, 0]
    g_hi = b2d[_BR - 1, 0]

    def seg(g, _):
        vals = jnp.where(b2d == g, h2, -1e30)
        m = jnp.max(vals, axis=0)
        cur = pooled_ref[pl.ds(g, 1), :]
        pooled_ref[pl.ds(g, 1), :] = jnp.maximum(cur, m[None, :])
        return 0
    lax.fori_loop(g_lo, g_hi + 1, seg, 0)

    @pl.when(i == _NB - 1)
    def _():
        pooled = pooled_ref[...]
        z = _dot(pooled, wa_ref[...]) + ba_ref[...]
        mu = jnp.mean(z, axis=-1, keepdims=True)
        dv = z - mu
        var = jnp.mean(dv * dv, axis=-1, keepdims=True)
        z = dv * lax.rsqrt(var + EPS) * g2_ref[...] + bb2_ref[...]
        z = jnp.where(z >= 0, z, 0.2 * z)
        z2 = _dot(z, wb_ref[...]) + bbb_ref[...]
        colid = lax.broadcasted_iota(jnp.int32, (G, 128), 1)
        valid = colid < C
        zm = jnp.where(valid, z2, -1e30)
        m = jnp.max(zm, axis=-1, keepdims=True)
        e = jnp.where(valid, jnp.exp(zm - m), 0.0)
        out_ref[...] = e / jnp.sum(e, axis=-1, keepdims=True)


def _row_spec():
    return pl.BlockSpec((_BR, H), lambda i: (i, 0))


def _row_spec_hi():
    return pl.BlockSpec((_BR, H), lambda i: (i + _NB, 0))


def _p_spec():
    return pl.BlockSpec((1, 2, _BR), lambda i: (i, 0, 0))


def _full2(shape):
    return pl.BlockSpec(shape, lambda i: (0,) * len(shape))


def kernel(x, edge_index, batch, W1, b1, ln1_g, ln1_b, W2, b2, Wa, ba,
           ln2_g, ln2_b, Wb, bb):
    row = edge_index[0]
    col = edge_index[1]

    hist2 = _hist_sc(col)
    p3 = hist2.reshape(_NB, 2, _BR)

    y1 = pl.pallas_call(
        _tc1_body,
        grid=(_NB,),
        in_specs=[_row_spec(), _full2((D, H)), _p_spec()],
        out_specs=_row_spec(),
        out_shape=jax.ShapeDtypeStruct((N, H), jnp.float32),
    )(x, W1, p3)

    s1p = _spmm_sc(row, col, y1)

    b1r = b1.reshape(1, H)
    g1r = ln1_g.reshape(1, H)
    bb1r = ln1_b.reshape(1, H)
    y2 = pl.pallas_call(
        _tc2_body,
        grid=(_NB,),
        in_specs=[_row_spec(), _row_spec_hi(), _row_spec(), _p_spec(),
                  _full2((1, H)), _full2((1, H)), _full2((1, H)),
                  _full2((H, H))],
        out_specs=_row_spec(),
        out_shape=jax.ShapeDtypeStruct((N, H), jnp.float32),
    )(s1p, s1p, y1, p3, b1r, g1r, bb1r, W2)

    s2p = _spmm_sc(row, col, y2)

    batch3 = jnp.broadcast_to(batch[:, None], (N, 8))
    Wbp = jnp.pad(Wb, ((0, 0), (0, 128 - C)))
    bbp = jnp.pad(bb, (0, 128 - C)).reshape(1, 128)
    out = pl.pallas_call(
        _tc3_body,
        grid=(_NB,),
        in_specs=[_row_spec(), _row_spec_hi(), _row_spec(), _p_spec(),
                  _full2((1, H)),
                  pl.BlockSpec((_BR, 8), lambda i: (i, 0)),
                  _full2((H, 768)), _full2((1, 768)), _full2((1, 768)),
                  _full2((1, 768)), _full2((768, 128)), _full2((1, 128))],
        out_specs=pl.BlockSpec((G, 128), lambda i: (0, 0)),
        out_shape=jax.ShapeDtypeStruct((G, 128), jnp.float32),
        scratch_shapes=[pltpu.VMEM((G, H), jnp.float32)],
    )(s2p, s2p, y2, p3, b2.reshape(1, H), batch3,
      Wa, ba.reshape(1, 768), ln2_g.reshape(1, 768), ln2_b.reshape(1, 768),
      Wbp, bbp)
    return out[:, :C]


# speedup vs baseline: 46.3207x; 1.0100x over previous
import functools

import jax
import jax.numpy as jnp
from jax import lax
from jax.experimental import pallas as pl
from jax.experimental.pallas import tpu as pltpu
from jax.experimental.pallas import tpu_sc as plsc

N = 10000
E = 320000
D = 128
H = 128
C = 10
G = 64
EPS = 1e-5

NC = 2
NS = 16
NW = NC * NS
EPW = E // NW
CHUNK = 40
NSTEP = EPW // CHUNK
NBK = 10
NOUT = NSTEP // NBK
NB = 5
KD = 4

def _mesh():
    return plsc.VectorSubcoreMesh(core_axis_name="c", subcore_axis_name="s")


def _zero_vmem2d(ref, nrows):
    def body(i, _):
        for j in range(8):
            ref[i, pl.ds(j * 16, 16)] = jnp.zeros((16,), jnp.float32)
        return 0
    lax.fori_loop(0, nrows, body, 0)


_HNI = 8
_HNS = 4


def _hist_body(col_hbm, out_hbm, ones_v, zbuf_v, acc_sh, *refs):
    c = lax.axis_index("c")
    s = lax.axis_index("s")
    wid = s * NC + c
    cv = refs[:_HNI]
    isem = refs[_HNI:2 * _HNI]
    ssem = refs[2 * _HNI:2 * _HNI + _HNS]
    ebase = wid * EPW

    def idx_load(t, j):
        pltpu.async_copy(col_hbm.at[pl.ds(ebase + t * CHUNK, CHUNK)],
                         cv[j], isem[j])

    def idx_wait(j):
        pltpu.make_async_copy(col_hbm.at[pl.ds(0, CHUNK)], cv[j],
                              isem[j]).wait()

    def s_start(j, r):
        pltpu.async_copy(ones_v, acc_sh.at[cv[j]], ssem[r], add=True)

    def s_wait(r):
        pltpu.make_async_copy(ones_v, acc_sh.at[cv[0]], ssem[r]).wait()

    for j in range(CHUNK // 16):
        ones_v[pl.ds(j * 16, 16)] = jnp.ones((16,), jnp.float32)
    if CHUNK % 16:
        ones_v[pl.ds(CHUNK - 16, 16)] = jnp.ones((16,), jnp.float32)
    for t in range(_HNS):
        idx_load(t, t)
    def zb(i, _):
        zbuf_v[pl.ds(i * 16, 16)] = jnp.zeros((16,), jnp.float32)
        return 0
    lax.fori_loop(0, 62, zb, 0)
    zbuf_v[pl.ds(984, 16)] = jnp.zeros((16,), jnp.float32)

    @pl.when(s < 10)
    def _():
        pltpu.sync_copy(zbuf_v, acc_sh.at[pl.ds(s * 1000, 1000)])
    plsc.subcore_barrier()

    def outer(i, _):
        for b in range(_HNI):
            t = i * _HNI + b

            @pl.when(t >= _HNS)
            def _():
                s_wait(b % _HNS)

            @pl.when(t + _HNS < NSTEP)
            def _():
                idx_load(t + _HNS, (b + _HNS) % _HNI)

            idx_wait(b)
            s_start(b, b % _HNS)
        return 0
    lax.fori_loop(0, NSTEP // _HNI, outer, 0)
    for t in range(NSTEP - NSTEP % _HNI, NSTEP):
        b = t % _HNI

        @pl.when(t >= _HNS)
        def _():
            s_wait(b % _HNS)

        idx_wait(b)
        s_start(b, b % _HNS)
    for r in range(_HNS):
        s_wait(r)
    plsc.subcore_barrier()

    @pl.when(s < 10)
    def _():
        pltpu.sync_copy(acc_sh.at[pl.ds(s * 1000, 1000)], zbuf_v)
        pltpu.sync_copy(zbuf_v, out_hbm.at[pl.ds(s * 2000 + c * 1000, 1000)])


def _hist_sc(col):
    k = pl.kernel(
        _hist_body,
        mesh=_mesh(),
        out_type=jax.ShapeDtypeStruct((2 * N,), jnp.float32),
        scratch_types=[
            pltpu.VMEM((CHUNK,), jnp.float32),
            pltpu.VMEM((1000,), jnp.float32),
            pltpu.VMEM_SHARED((N,), jnp.float32),
        ] + [pltpu.VMEM((CHUNK,), jnp.int32)] * _HNI
          + [pltpu.SemaphoreType.DMA] * (_HNI + _HNS),
    )
    return k(col)


NI = 10


def _spmm_body(row_hbm, col_hbm, y_hbm, out_hbm, bufs, acc_sh, *refs):
    c = lax.axis_index("c")
    s = lax.axis_index("s")
    wid = s * NC + c
    rv = refs[:NI]
    cv = refs[NI:2 * NI]
    isem = refs[2 * NI:3 * NI]
    gsem = refs[3 * NI:3 * NI + NB]
    ssem = refs[3 * NI + NB:3 * NI + 2 * NB]
    ebase = wid * EPW

    def bufref(b):
        return bufs.at[pl.ds(b * CHUNK, CHUNK)]

    def idx_load(t, j):
        pltpu.async_copy(row_hbm.at[pl.ds(ebase + t * CHUNK, CHUNK)],
                         rv[j], isem[j])
        pltpu.async_copy(col_hbm.at[pl.ds(ebase + t * CHUNK, CHUNK)],
                         cv[j], isem[j])

    def idx_wait(j):
        pltpu.make_async_copy(row_hbm.at[pl.ds(0, CHUNK)], rv[j],
                              isem[j]).wait()
        pltpu.make_async_copy(col_hbm.at[pl.ds(0, CHUNK)], cv[j],
                              isem[j]).wait()

    def g_start(j, b):
        pltpu.async_copy(y_hbm.at[rv[j]], bufref(b), gsem[b])

    def g_wait(b):
        pltpu.make_async_copy(y_hbm.at[rv[0]], bufref(b), gsem[b]).wait()

    def s_start(j, b):
        pltpu.async_copy(bufref(b), acc_sh.at[cv[j]], ssem[b], add=True)

    def s_wait(b):
        pltpu.make_async_copy(bufref(b), acc_sh.at[cv[0]], ssem[b]).wait()

    for t in range(NI - 3):
        idx_load(t, t)
    _zero_vmem2d(bufs, NB * CHUNK)
    @pl.when(s < 10)
    def _():
        for k in range(5):
            pltpu.sync_copy(bufs, acc_sh.at[pl.ds(s * 1000 + k * 200, 200)])
    plsc.subcore_barrier()

    for t in range(KD):
        idx_wait(t)
        g_start(t, t)

    def outer(i, _):
        for b in range(NBK):
            t = i * NBK + b
            bb = b % NB
            bg = (b + KD) % NB

            @pl.when(t + NI - 3 < NSTEP)
            def _():
                idx_load(t + NI - 3, (b + NI - 3) % NI)

            @pl.when(t >= NB - KD)
            def _():
                s_wait(bg)

            @pl.when(t + KD < NSTEP)
            def _():
                idx_wait((b + KD) % NI)
                g_start((b + KD) % NI, bg)

            g_wait(bb)
            s_start(b, bb)
        return 0
    lax.fori_loop(0, NOUT, outer, 0)

    for t in range(NSTEP - (NB - KD), NSTEP):
        s_wait(t % NB)
    plsc.subcore_barrier()

    @pl.when(s < 10)
    def _():
        for k in range(5):
            r0 = s * 1000 + k * 200
            pltpu.sync_copy(acc_sh.at[pl.ds(r0, 200)], bufs)
            pltpu.sync_copy(bufs, out_hbm.at[pl.ds(c * N + r0, 200)])


def _spmm_sc(row, col, y):
    k = pl.kernel(
        _spmm_body,
        mesh=_mesh(),
        out_type=jax.ShapeDtypeStruct((2 * N, H), jnp.float32),
        scratch_types=[
            pltpu.VMEM((NB * CHUNK, H), jnp.float32),
            pltpu.VMEM_SHARED((N, H), jnp.float32),
        ] + [pltpu.VMEM((CHUNK,), jnp.int32)] * (2 * NI)
          + [pltpu.SemaphoreType.DMA] * (NI + 2 * NB),
    )
    return k(row, col, y)


_BR = 1000
_NB = N // _BR


def _dot(a, b):
    return jnp.dot(a, b, preferred_element_type=jnp.float32,
                   precision=lax.Precision.HIGHEST)


def _tc0_body(x_ref, w1_ref, xw_ref):
    xw_ref[...] = _dot(x_ref[...], w1_ref[...])


def _tc1_body(xw_ref, p_ref, y1_ref):
    hist = p_ref[0, 0, :] + p_ref[0, 1, :]
    dis = lax.rsqrt(hist + 2.0)
    y1_ref[...] = xw_ref[...] * dis[:, None]


def _tc2_body(s0_ref, s1_ref, y1_ref, p_ref, b1_ref, g1_ref,
              bb1_ref, w2_ref, y2_ref):
    hist = p_ref[0, 0, :] + p_ref[0, 1, :]
    dis1 = lax.rsqrt(hist + 2.0)
    dis2 = lax.rsqrt(hist + 1.0)
    srow = s0_ref[...] + s1_ref[...]
    pre = (srow * dis1[:, None]
           + (2.0 * dis1)[:, None] * y1_ref[...]
           + b1_ref[...])
    mu = jnp.mean(pre, axis=-1, keepdims=True)
    dvar = pre - mu
    var = jnp.mean(dvar * dvar, axis=-1, keepdims=True)
    t = dvar * lax.rsqrt(var + EPS) * g1_ref[...] + bb1_ref[...]
    h1 = jnp.where(t >= 0, t, 0.2 * t)
    xw2 = _dot(h1, w2_ref[...])
    y2_ref[...] = xw2 * dis2[:, None]


def _tc3_body(s0_ref, s1_ref, y2_ref, p_ref, b2_ref, batch_ref,
              wa_ref, ba_ref, g2_ref, bb2_ref, wb_ref, bbb_ref,
              out_ref, pooled_ref):
    i = pl.program_id(0)

    @pl.when(i == 0)
    def _():
        pooled_ref[...] = jnp.full((G, H), -1e30, jnp.float32)

    hist = p_ref[0, 0, :] + p_ref[0, 1, :]
    dis2 = lax.rsqrt(hist + 1.0)
    srow = s0_ref[...] + s1_ref[...]
    t = (srow * dis2[:, None]
         + dis2[:, None] * y2_ref[...]
         + b2_ref[...])
    h2 = jnp.where(t >= 0, t, 0.2 * t)

    b2d = batch_ref[...][:, 0:1]
    g_lo = b2d[0, 0]
    g_hi = b2d[_BR - 1, 0]

    def seg(g, _):
        vals = jnp.where(b2d == g, h2, -1e30)
        m = jnp.max(vals, axis=0)
        cur = pooled_ref[pl.ds(g, 1), :]
        pooled_ref[pl.ds(g, 1), :] = jnp.maximum(cur, m[None, :])
        return 0
    lax.fori_loop(g_lo, g_hi + 1, seg, 0)

    @pl.when(i == _NB - 1)
    def _():
        pooled = pooled_ref[...]
        z = _dot(pooled, wa_ref[...]) + ba_ref[...]
        mu = jnp.mean(z, axis=-1, keepdims=True)
        dv = z - mu
        var = jnp.mean(dv * dv, axis=-1, keepdims=True)
        z = dv * lax.rsqrt(var + EPS) * g2_ref[...] + bb2_ref[...]
        z = jnp.where(z >= 0, z, 0.2 * z)
        z2 = _dot(z, wb_ref[...]) + bbb_ref[...]
        colid = lax.broadcasted_iota(jnp.int32, (G, 128), 1)
        valid = colid < C
        zm = jnp.where(valid, z2, -1e30)
        m = jnp.max(zm, axis=-1, keepdims=True)
        e = jnp.where(valid, jnp.exp(zm - m), 0.0)
        out_ref[...] = e / jnp.sum(e, axis=-1, keepdims=True)


def _row_spec():
    return pl.BlockSpec((_BR, H), lambda i: (i, 0))


def _row_spec_hi():
    return pl.BlockSpec((_BR, H), lambda i: (i + _NB, 0))


def _p_spec():
    return pl.BlockSpec((1, 2, _BR), lambda i: (i, 0, 0))


def _full2(shape):
    return pl.BlockSpec(shape, lambda i: (0,) * len(shape))


def kernel(x, edge_index, batch, W1, b1, ln1_g, ln1_b, W2, b2, Wa, ba,
           ln2_g, ln2_b, Wb, bb):
    row = edge_index[0]
    col = edge_index[1]

    hist2 = _hist_sc(col)
    p3 = hist2.reshape(_NB, 2, _BR)

    xw1 = pl.pallas_call(
        _tc0_body,
        grid=(_NB,),
        in_specs=[_row_spec(), _full2((D, H))],
        out_specs=_row_spec(),
        out_shape=jax.ShapeDtypeStruct((N, H), jnp.float32),
    )(x, W1)

    y1 = pl.pallas_call(
        _tc1_body,
        grid=(_NB,),
        in_specs=[_row_spec(), _p_spec()],
        out_specs=_row_spec(),
        out_shape=jax.ShapeDtypeStruct((N, H), jnp.float32),
    )(xw1, p3)

    s1p = _spmm_sc(row, col, y1)

    b1r = b1.reshape(1, H)
    g1r = ln1_g.reshape(1, H)
    bb1r = ln1_b.reshape(1, H)
    y2 = pl.pallas_call(
        _tc2_body,
        grid=(_NB,),
        in_specs=[_row_spec(), _row_spec_hi(), _row_spec(), _p_spec(),
                  _full2((1, H)), _full2((1, H)), _full2((1, H)),
                  _full2((H, H))],
        out_specs=_row_spec(),
        out_shape=jax.ShapeDtypeStruct((N, H), jnp.float32),
    )(s1p, s1p, y1, p3, b1r, g1r, bb1r, W2)

    s2p = _spmm_sc(row, col, y2)

    batch3 = jnp.broadcast_to(batch[:, None], (N, 8))
    Wbp = jnp.pad(Wb, ((0, 0), (0, 128 - C)))
    bbp = jnp.pad(bb, (0, 128 - C)).reshape(1, 128)
    out = pl.pallas_call(
        _tc3_body,
        grid=(_NB,),
        in_specs=[_row_spec(), _row_spec_hi(), _row_spec(), _p_spec(),
                  _full2((1, H)),
                  pl.BlockSpec((_BR, 8), lambda i: (i, 0)),
                  _full2((H, 768)), _full2((1, 768)), _full2((1, 768)),
                  _full2((1, 768)), _full2((768, 128)), _full2((1, 128))],
        out_specs=pl.BlockSpec((G, 128), lambda i: (0, 0)),
        out_shape=jax.ShapeDtypeStruct((G, 128), jnp.float32),
        scratch_shapes=[pltpu.VMEM((G, H), jnp.float32)],
    )(s2p, s2p, y2, p3, b2.reshape(1, H), batch3,
      Wa, ba.reshape(1, 768), ln2_g.reshape(1, 768), ln2_b.reshape(1, 768),
      Wbp, bbp)
    return out[:, :C]
